# Initial kernel scaffold; baseline (speedup 1.0000x reference)
import jax
import jax.numpy as jnp
from jax.experimental import pallas as pl


def kernel(x_categorical, x_numerical, tables, bn0_g, bn0_b, W0, b0, g0, beta0, W1, b1, g1, beta1, W2, b2, g2, beta2, W3, b3, g3, beta3, Wout, bout):
    raise NotImplementedError("write your pallas kernel here")


# speedup vs baseline: 7.2431x; 7.2431x over previous
import functools

import jax
import jax.numpy as jnp
from jax import lax
from jax.experimental import pallas as pl
from jax.experimental.pallas import tpu as pltpu
from jax.experimental.pallas import tpu_sc as plsc

B = 16384
NCAT = 26
V = 100000
D = 16
NNUM = 13
EPS = 1e-5

_NC = 2
_NS = 16
_NW = _NC * _NS

_TOTAL = B * NCAT
_PER_W = _TOTAL // _NW
_CHUNK = 1664
_NCHUNK = _PER_W // _CHUNK


def _sc_gather(flat_idx, flat_tables):
    mesh = plsc.VectorSubcoreMesh(core_axis_name="c", subcore_axis_name="s")

    @functools.partial(
        pl.kernel,
        mesh=mesh,
        compiler_params=pltpu.CompilerParams(use_tc_tiling_on_sc=False),
        out_type=jax.ShapeDtypeStruct((_TOTAL, D), jnp.float32),
        scratch_types=[
            pltpu.VMEM((_CHUNK,), jnp.int32),
            pltpu.VMEM((_CHUNK, D), jnp.float32),
            pltpu.VMEM((_CHUNK,), jnp.int32),
            pltpu.VMEM((_CHUNK, D), jnp.float32),
            pltpu.SemaphoreType.DMA,
            pltpu.SemaphoreType.DMA,
        ],
    )
    def gather_kernel(idx_hbm, tab_hbm, out_hbm, idx0, rows0, idx1, rows1,
                      sem0, sem1):
        wid = lax.axis_index("s") * _NC + lax.axis_index("c")
        base = pl.multiple_of(wid * _PER_W, 8)

        idx_bufs = (idx0, idx1)
        row_bufs = (rows0, rows1)
        sems = (sem0, sem1)

        def load_and_fire(i):
            slot = i % 2
            pltpu.sync_copy(idx_hbm.at[pl.ds(base + i * _CHUNK, _CHUNK)],
                            idx_bufs[slot])
            return pltpu.async_copy(tab_hbm.at[idx_bufs[slot]],
                                    row_bufs[slot], sems[slot])

        cps = [load_and_fire(0)]
        for i in range(_NCHUNK):
            if i + 1 < _NCHUNK:
                cps.append(load_and_fire(i + 1))
            cps[i].wait()
            pltpu.sync_copy(row_bufs[i % 2],
                            out_hbm.at[pl.ds(base + i * _CHUNK, _CHUNK)])

    return gather_kernel(flat_idx, flat_tables)


def _leaky(t):
    return jnp.where(t >= 0, t, 0.01 * t)


def _layer_body(x_ref, a_ref, c_ref, w_ref, b_ref, y_ref, s_ref, q_ref):
    xn = x_ref[...] * a_ref[...] + c_ref[...]
    t = jnp.dot(xn, w_ref[...], preferred_element_type=jnp.float32) + b_ref[...]
    y = _leaky(t)
    y_ref[...] = y

    @pl.when(pl.program_id(0) == 0)
    def _():
        s_ref[...] = jnp.zeros_like(s_ref)
        q_ref[...] = jnp.zeros_like(q_ref)

    s_ref[...] += jnp.sum(y, axis=0, keepdims=True)
    q_ref[...] += jnp.sum(y * y, axis=0, keepdims=True)


def _layer(x, a, c, w, b, tb):
    bb, hp = x.shape
    h = w.shape[1]
    grid = (bb // tb,)
    return pl.pallas_call(
        _layer_body,
        grid=grid,
        in_specs=[
            pl.BlockSpec((tb, hp), lambda i: (i, 0)),
            pl.BlockSpec((1, hp), lambda i: (0, 0)),
            pl.BlockSpec((1, hp), lambda i: (0, 0)),
            pl.BlockSpec((hp, h), lambda i: (0, 0)),
            pl.BlockSpec((1, h), lambda i: (0, 0)),
        ],
        out_specs=[
            pl.BlockSpec((tb, h), lambda i: (i, 0)),
            pl.BlockSpec((1, h), lambda i: (0, 0)),
            pl.BlockSpec((1, h), lambda i: (0, 0)),
        ],
        out_shape=[
            jax.ShapeDtypeStruct((bb, h), jnp.float32),
            jax.ShapeDtypeStruct((1, h), jnp.float32),
            jax.ShapeDtypeStruct((1, h), jnp.float32),
        ],
    )(x, a.reshape(1, hp), c.reshape(1, hp), w, b.reshape(1, h))


def _layer0_body(e_ref, x_ref, a_ref, c_ref, we_ref, wn_ref, b_ref,
                 y_ref, s_ref, q_ref):
    xn = x_ref[...] * a_ref[...] + c_ref[...]
    t = jnp.dot(e_ref[...], we_ref[...], preferred_element_type=jnp.float32)
    t += jnp.dot(xn, wn_ref[...], preferred_element_type=jnp.float32)
    t += b_ref[...]
    y = _leaky(t)
    y_ref[...] = y

    @pl.when(pl.program_id(0) == 0)
    def _():
        s_ref[...] = jnp.zeros_like(s_ref)
        q_ref[...] = jnp.zeros_like(q_ref)

    s_ref[...] += jnp.sum(y, axis=0, keepdims=True)
    q_ref[...] += jnp.sum(y * y, axis=0, keepdims=True)


def _layer0(emb, xnum, a, c, w_emb, w_num, b, tb):
    bb, he = emb.shape
    hn = xnum.shape[1]
    h = w_emb.shape[1]
    grid = (bb // tb,)
    return pl.pallas_call(
        _layer0_body,
        grid=grid,
        in_specs=[
            pl.BlockSpec((tb, he), lambda i: (i, 0)),
            pl.BlockSpec((tb, hn), lambda i: (i, 0)),
            pl.BlockSpec((1, hn), lambda i: (0, 0)),
            pl.BlockSpec((1, hn), lambda i: (0, 0)),
            pl.BlockSpec((he, h), lambda i: (0, 0)),
            pl.BlockSpec((hn, h), lambda i: (0, 0)),
            pl.BlockSpec((1, h), lambda i: (0, 0)),
        ],
        out_specs=[
            pl.BlockSpec((tb, h), lambda i: (i, 0)),
            pl.BlockSpec((1, h), lambda i: (0, 0)),
            pl.BlockSpec((1, h), lambda i: (0, 0)),
        ],
        out_shape=[
            jax.ShapeDtypeStruct((bb, h), jnp.float32),
            jax.ShapeDtypeStruct((1, h), jnp.float32),
            jax.ShapeDtypeStruct((1, h), jnp.float32),
        ],
    )(emb, xnum, a.reshape(1, hn), c.reshape(1, hn), w_emb, w_num,
      b.reshape(1, h))


def _stats_body(x_ref, s_ref, q_ref):
    x = x_ref[...]

    @pl.when(pl.program_id(0) == 0)
    def _():
        s_ref[...] = jnp.zeros_like(s_ref)
        q_ref[...] = jnp.zeros_like(q_ref)

    s_ref[...] += jnp.sum(x, axis=0, keepdims=True)
    q_ref[...] += jnp.sum(x * x, axis=0, keepdims=True)


def _stats(x, tb):
    bb, h = x.shape
    return pl.pallas_call(
        _stats_body,
        grid=(bb // tb,),
        in_specs=[pl.BlockSpec((tb, h), lambda i: (i, 0))],
        out_specs=[
            pl.BlockSpec((1, h), lambda i: (0, 0)),
            pl.BlockSpec((1, h), lambda i: (0, 0)),
        ],
        out_shape=[
            jax.ShapeDtypeStruct((1, h), jnp.float32),
            jax.ShapeDtypeStruct((1, h), jnp.float32),
        ],
    )(x)


def _final_body(x_ref, a_ref, c_ref, w_ref, b_ref, o_ref):
    xn = x_ref[...] * a_ref[...] + c_ref[...]
    o_ref[...] = (jnp.dot(xn, w_ref[...], preferred_element_type=jnp.float32)
                  + b_ref[...])


def _final(x, a, c, w, b, tb):
    bb, hp = x.shape
    h = w.shape[1]
    return pl.pallas_call(
        _final_body,
        grid=(bb // tb,),
        in_specs=[
            pl.BlockSpec((tb, hp), lambda i: (i, 0)),
            pl.BlockSpec((1, hp), lambda i: (0, 0)),
            pl.BlockSpec((1, hp), lambda i: (0, 0)),
            pl.BlockSpec((hp, h), lambda i: (0, 0)),
            pl.BlockSpec((1, h), lambda i: (0, 0)),
        ],
        out_specs=pl.BlockSpec((tb, h), lambda i: (i, 0)),
        out_shape=jax.ShapeDtypeStruct((bb, h), jnp.float32),
    )(x, a.reshape(1, hp), c.reshape(1, hp), w, b.reshape(1, h))


def _bn_coeffs(s, q, gamma, beta):
    mu = s.reshape(-1) / B
    var = jnp.maximum(q.reshape(-1) / B - mu * mu, 0.0)
    a = gamma * lax.rsqrt(var + EPS)
    return a, beta - mu * a


def kernel(x_categorical, x_numerical, tables, bn0_g, bn0_b,
           W0, b0, g0, beta0, W1, b1, g1, beta1,
           W2, b2, g2, beta2, W3, b3, g3, beta3, Wout, bout):
    flat_idx = (x_categorical
                + (jnp.arange(NCAT, dtype=jnp.int32) * V)[None, :]).reshape(-1)
    flat_tables = tables.reshape(NCAT * V, D)

    emb = _sc_gather(flat_idx, flat_tables).reshape(B, NCAT * D)

    s0, q0 = _stats(x_numerical, 2048)
    a0, c0 = _bn_coeffs(s0, q0, bn0_g, bn0_b)

    tb = 1024
    y0, s, q = _layer0(emb, x_numerical, a0, c0,
                       W0[: NCAT * D], W0[NCAT * D:], b0, tb)
    a, c = _bn_coeffs(s, q, g0, beta0)
    y1, s, q = _layer(y0, a, c, W1, b1, tb)
    a, c = _bn_coeffs(s, q, g1, beta1)
    y2, s, q = _layer(y1, a, c, W2, b2, tb)
    a, c = _bn_coeffs(s, q, g2, beta2)
    y3, s, q = _layer(y2, a, c, W3, b3, tb)
    a, c = _bn_coeffs(s, q, g3, beta3)
    return _final(y3, a, c, Wout, bout, tb)


# speedup vs baseline: 7.4705x; 1.0314x over previous
import functools

import jax
import jax.numpy as jnp
from jax import lax
from jax.experimental import pallas as pl
from jax.experimental.pallas import tpu as pltpu
from jax.experimental.pallas import tpu_sc as plsc

B = 16384
NCAT = 26
V = 100000
D = 16
NNUM = 13
EMB = NCAT * D
EPS = 1e-5

_NC = 2
_NS = 16
_NW = _NC * _NS

_TOTAL = B * NCAT
_PER_W = _TOTAL // _NW
_CHUNK = 1664
_NCHUNK = _PER_W // _CHUNK

TB0 = 1024
TB = 512
NT = B // TB


def _sc_gather(flat_idx, flat_tables):
    mesh = plsc.VectorSubcoreMesh(core_axis_name="c", subcore_axis_name="s")

    @functools.partial(
        pl.kernel,
        mesh=mesh,
        compiler_params=pltpu.CompilerParams(use_tc_tiling_on_sc=False),
        out_type=jax.ShapeDtypeStruct((_TOTAL, D), jnp.float32),
        scratch_types=[
            pltpu.VMEM((_CHUNK,), jnp.int32),
            pltpu.VMEM((_CHUNK, D), jnp.float32),
            pltpu.VMEM((_CHUNK,), jnp.int32),
            pltpu.VMEM((_CHUNK, D), jnp.float32),
            pltpu.SemaphoreType.DMA,
            pltpu.SemaphoreType.DMA,
        ],
    )
    def gather_kernel(idx_hbm, tab_hbm, out_hbm, idx0, rows0, idx1, rows1,
                      sem0, sem1):
        wid = lax.axis_index("s") * _NC + lax.axis_index("c")
        base = pl.multiple_of(wid * _PER_W, 8)

        idx_bufs = (idx0, idx1)
        row_bufs = (rows0, rows1)
        sems = (sem0, sem1)

        def load_and_fire(i):
            slot = i % 2
            pltpu.sync_copy(idx_hbm.at[pl.ds(base + i * _CHUNK, _CHUNK)],
                            idx_bufs[slot])
            return pltpu.async_copy(tab_hbm.at[idx_bufs[slot]],
                                    row_bufs[slot], sems[slot])

        cps = [load_and_fire(0)]
        for i in range(_NCHUNK):
            if i + 1 < _NCHUNK:
                cps.append(load_and_fire(i + 1))
            cps[i].wait()
            pltpu.sync_copy(row_bufs[i % 2],
                            out_hbm.at[pl.ds(base + i * _CHUNK, _CHUNK)])

    return gather_kernel(flat_idx, flat_tables)


def _leaky(t):
    return jnp.where(t >= 0, t, 0.01 * t)


def _bn_ac(s, q, gamma, beta):
    mu = s * (1.0 / B)
    var = jnp.maximum(q * (1.0 / B) - mu * mu, 0.0)
    a = gamma * lax.rsqrt(var + EPS)
    return a, beta - mu * a


def _stats_body(x_ref, s_ref, q_ref):
    x = x_ref[...]

    @pl.when(pl.program_id(0) == 0)
    def _():
        s_ref[...] = jnp.zeros_like(s_ref)
        q_ref[...] = jnp.zeros_like(q_ref)

    s_ref[...] += jnp.sum(x, axis=0, keepdims=True)
    q_ref[...] += jnp.sum(x * x, axis=0, keepdims=True)


def _stats(x, tb):
    bb, h = x.shape
    return pl.pallas_call(
        _stats_body,
        grid=(bb // tb,),
        in_specs=[pl.BlockSpec((tb, h), lambda i: (i, 0))],
        out_specs=[
            pl.BlockSpec((1, h), lambda i: (0, 0)),
            pl.BlockSpec((1, h), lambda i: (0, 0)),
        ],
        out_shape=[
            jax.ShapeDtypeStruct((1, h), jnp.float32),
            jax.ShapeDtypeStruct((1, h), jnp.float32),
        ],
    )(x)


def _layer0_body(e_ref, x_ref, s0_ref, q0_ref, g_ref, bt_ref,
                 we_ref, wn_ref, b_ref, y_ref, s_ref, q_ref):
    a0, c0 = _bn_ac(s0_ref[...], q0_ref[...], g_ref[...], bt_ref[...])
    xn = x_ref[...] * a0 + c0
    t = jnp.dot(e_ref[...], we_ref[...], preferred_element_type=jnp.float32)
    t += jnp.dot(xn, wn_ref[...], preferred_element_type=jnp.float32)
    y = _leaky(t + b_ref[...])
    y_ref[...] = y

    @pl.when(pl.program_id(0) == 0)
    def _():
        s_ref[...] = jnp.zeros_like(s_ref)
        q_ref[...] = jnp.zeros_like(q_ref)

    s_ref[...] += jnp.sum(y, axis=0, keepdims=True)
    q_ref[...] += jnp.sum(y * y, axis=0, keepdims=True)


def _layer0(emb, xnum, s0, q0, bn0g, bn0b, w_emb, w_num, b):
    h = w_emb.shape[1]
    row = lambda v: v.reshape(1, -1)
    const = lambda shape: pl.BlockSpec(shape, lambda i: (0, 0))
    return pl.pallas_call(
        _layer0_body,
        grid=(B // TB0,),
        in_specs=[
            pl.BlockSpec((TB0, EMB), lambda i: (i, 0)),
            pl.BlockSpec((TB0, NNUM), lambda i: (i, 0)),
            const((1, NNUM)), const((1, NNUM)), const((1, NNUM)),
            const((1, NNUM)),
            const((EMB, h)), const((NNUM, h)), const((1, h)),
        ],
        out_specs=[
            pl.BlockSpec((TB0, h), lambda i: (i, 0)),
            const((1, h)),
            const((1, h)),
        ],
        out_shape=[
            jax.ShapeDtypeStruct((B, h), jnp.float32),
            jax.ShapeDtypeStruct((1, h), jnp.float32),
            jax.ShapeDtypeStruct((1, h), jnp.float32),
        ],
    )(emb, xnum, s0, q0, row(bn0g), row(bn0b), w_emb, w_num, row(b))


def _tail_body(y0_hbm, s_in, q_in,
               g0, bt0, w1, b1, g1, bt1,
               w2, b2, g2, bt2, w3, b3, g3, bt3,
               wout, bout,
               out_hbm,
               ybuf, obuf, y1, y2, y3,
               ysem, osem):

    def y0_load(i):
        return pltpu.make_async_copy(
            y0_hbm.at[pl.ds(i * TB, TB)], ybuf.at[i % 2], ysem.at[i % 2])

    a1, c1 = _bn_ac(s_in[...], q_in[...], g0[...], bt0[...])

    y0_load(0).start()
    s = jnp.zeros((1, 300), jnp.float32)
    q = jnp.zeros((1, 300), jnp.float32)
    for i in range(NT):
        if i + 1 < NT:
            y0_load(i + 1).start()
        y0_load(i).wait()
        x = ybuf[i % 2] * a1 + c1
        t = jnp.dot(x, w1[...], preferred_element_type=jnp.float32)
        y = _leaky(t + b1[...])
        y1[pl.ds(i * TB, TB), :] = y
        s = s + jnp.sum(y, axis=0, keepdims=True)
        q = q + jnp.sum(y * y, axis=0, keepdims=True)

    def layer(y_prev, y_next, w, b, a, c, h):
        s = jnp.zeros((1, h), jnp.float32)
        q = jnp.zeros((1, h), jnp.float32)
        for i in range(NT):
            x = y_prev[pl.ds(i * TB, TB), :] * a + c
            t = jnp.dot(x, w[...], preferred_element_type=jnp.float32)
            y = _leaky(t + b[...])
            y_next[pl.ds(i * TB, TB), :] = y
            s = s + jnp.sum(y, axis=0, keepdims=True)
            q = q + jnp.sum(y * y, axis=0, keepdims=True)
        return s, q

    a, c = _bn_ac(s, q, g1[...], bt1[...])
    s, q = layer(y1, y2, w2, b2, a, c, 200)
    a, c = _bn_ac(s, q, g2[...], bt2[...])
    s, q = layer(y2, y3, w3, b3, a, c, 100)
    a, c = _bn_ac(s, q, g3[...], bt3[...])

    def o_copy(i):
        return pltpu.make_async_copy(
            obuf.at[i % 2], out_hbm.at[pl.ds(i * TB, TB)], osem.at[i % 2])

    for i in range(NT):
        if i >= 2:
            o_copy(i - 2).wait()
        x = y3[pl.ds(i * TB, TB), :] * a + c
        obuf[i % 2] = (jnp.dot(x, wout[...], preferred_element_type=jnp.float32)
                       + bout[...])
        o_copy(i).start()
    o_copy(NT - 2).wait()
    o_copy(NT - 1).wait()


def _tail(y0, s, q, g0, bt0, w1, b1, g1, bt1,
          w2, b2, g2, bt2, w3, b3, g3, bt3, wout, bout):
    hbm = pl.BlockSpec(memory_space=pltpu.MemorySpace.HBM)
    vmem = pl.BlockSpec(memory_space=pltpu.MemorySpace.VMEM)
    row = lambda v: v.reshape(1, -1)
    return pl.pallas_call(
        _tail_body,
        in_specs=[hbm] + [vmem] * 18,
        out_specs=hbm,
        out_shape=jax.ShapeDtypeStruct((B, 1), jnp.float32),
        scratch_shapes=[
            pltpu.VMEM((2, TB, 600), jnp.float32),
            pltpu.VMEM((2, TB, 1), jnp.float32),
            pltpu.VMEM((B, 300), jnp.float32),
            pltpu.VMEM((B, 200), jnp.float32),
            pltpu.VMEM((B, 100), jnp.float32),
            pltpu.SemaphoreType.DMA((2,)),
            pltpu.SemaphoreType.DMA((2,)),
        ],
        compiler_params=pltpu.CompilerParams(
            vmem_limit_bytes=63 * 1024 * 1024),
    )(y0, s, q, row(g0), row(bt0), w1, row(b1), row(g1), row(bt1),
      w2, row(b2), row(g2), row(bt2), w3, row(b3), row(g3), row(bt3),
      wout, row(bout))


def kernel(x_categorical, x_numerical, tables, bn0_g, bn0_b,
           W0, b0, g0, beta0, W1, b1, g1, beta1,
           W2, b2, g2, beta2, W3, b3, g3, beta3, Wout, bout):
    flat_idx = (x_categorical
                + (jnp.arange(NCAT, dtype=jnp.int32) * V)[None, :]).reshape(-1)
    flat_tables = tables.reshape(NCAT * V, D)

    emb = _sc_gather(flat_idx, flat_tables).reshape(B, EMB)

    s0, q0 = _stats(x_numerical, 2048)
    y0, s, q = _layer0(emb, x_numerical, s0, q0, bn0_g, bn0_b,
                       W0[:EMB], W0[EMB:], b0)
    return _tail(y0, s, q, g0, beta0, W1, b1, g1, beta1,
                 W2, b2, g2, beta2, W3, b3, g3, beta3, Wout, bout)
